# Initial kernel scaffold; baseline (speedup 1.0000x reference)
import jax
import jax.numpy as jnp
from jax.experimental import pallas as pl


def kernel(actions, We1, be1, We2, be2, We3, be3, codebook, Wd1, bd1, Wd2, bd2, Wd3, bd3):
    raise NotImplementedError("write your pallas kernel here")


# speedup vs baseline: 1.1524x; 1.1524x over previous
import functools

import jax
import jax.numpy as jnp
from jax.experimental import pallas as pl

B, T, A = 4096, 50, 3
H, D, N, K = 256, 32, 4, 8192
FLAT = T * A
CC = 0.25
BT = 256


def _silu(x):
    return x * jax.nn.sigmoid(x)


def _mm_bf16(a, b):
    return jnp.dot(a.astype(jnp.bfloat16), b.astype(jnp.bfloat16),
                   preferred_element_type=jnp.float32)


def _mm_f32(a, b):
    return jnp.dot(a, b, preferred_element_type=jnp.float32)


def _vq_kernel(x_ref, We1_ref, be1_ref, We2_ref, be2_ref, We3_ref, be3_ref,
               cb_ref, cbT_ref, Wd1_ref, bd1_ref, Wd2_ref, bd2_ref, Wd3_ref,
               bd3_ref, recon_ref, idx_ref, vq_ref, rec_ref):
    i = pl.program_id(0)
    x = x_ref[...]
    h = _silu(_mm_bf16(x, We1_ref[...]) + be1_ref[...])
    h = _silu(_mm_bf16(h, We2_ref[...]) + be2_ref[...])
    enc = _mm_bf16(h, We3_ref[...]) + be3_ref[...]

    cbT = cbT_ref[...]
    cn = jnp.sum(cbT * cbT, axis=0, keepdims=True)
    iota_k = jax.lax.broadcasted_iota(jnp.int32, (1, K), 1)
    KH = K // 2
    iota_h = jax.lax.broadcasted_iota(jnp.int32, (1, KH), 1)

    idxs = []
    quants = []
    for n in range(N):
        fln = enc[:, n * D:(n + 1) * D]
        ln = jnp.sum(fln * fln, axis=1, keepdims=True)
        m = _mm_f32(fln, cbT)
        dist = (ln + cn) - 2.0 * m
        dA = dist[:, :KH]
        dB = dist[:, KH:]
        mnA = jnp.min(dA, axis=1, keepdims=True)
        amA = jnp.min(jnp.where(dA == mnA, iota_h, K), axis=1, keepdims=True)
        mnB = jnp.min(dB, axis=1, keepdims=True)
        amB = jnp.min(jnp.where(dB == mnB, iota_h, K), axis=1,
                      keepdims=True) + KH
        accA = mnA.astype(jnp.bfloat16).astype(jnp.float32)
        amin = jnp.where(mnB < accA, amB, amA)
        oh = (iota_k == amin).astype(jnp.float32)
        qn = _mm_f32(oh, cb_ref[...])
        idxs.append(amin)
        quants.append(qn)

    idx_ref[...] = jnp.concatenate(idxs, axis=1)
    quant = jnp.concatenate(quants, axis=1)
    diff = quant - enc
    vq_part = jnp.sum(diff * diff)

    dec = _silu(_mm_bf16(quant, Wd1_ref[...]) + bd1_ref[...])
    dec = _silu(_mm_bf16(dec, Wd2_ref[...]) + bd2_ref[...])
    recon = _mm_bf16(dec, Wd3_ref[...]) + bd3_ref[...]
    recon_ref[...] = recon
    rdiff = recon - x
    rec_part = jnp.sum(rdiff * rdiff)

    @pl.when(i == 0)
    def _init():
        vq_ref[...] = jnp.zeros((1, 1), jnp.float32)
        rec_ref[...] = jnp.zeros((1, 1), jnp.float32)

    vq_ref[...] += vq_part
    rec_ref[...] += rec_part


def kernel(actions, We1, be1, We2, be2, We3, be3, codebook, Wd1, bd1, Wd2,
           bd2, Wd3, bd3):
    x2d = actions.reshape(B, FLAT)
    cbT = codebook.T
    row = lambda v: v.reshape(1, -1)

    full = lambda shape: pl.BlockSpec(shape, lambda i: (0, 0))
    grid = (B // BT,)
    recon2d, idx, vq_sum, rec_sum = pl.pallas_call(
        _vq_kernel,
        grid=grid,
        in_specs=[
            pl.BlockSpec((BT, FLAT), lambda i: (i, 0)),
            full((FLAT, H)), full((1, H)),
            full((H, H)), full((1, H)),
            full((H, N * D)), full((1, N * D)),
            full((K, D)), full((D, K)),
            full((N * D, H)), full((1, H)),
            full((H, H)), full((1, H)),
            full((H, FLAT)), full((1, FLAT)),
        ],
        out_specs=[
            pl.BlockSpec((BT, FLAT), lambda i: (i, 0)),
            pl.BlockSpec((BT, N), lambda i: (i, 0)),
            pl.BlockSpec((1, 1), lambda i: (0, 0)),
            pl.BlockSpec((1, 1), lambda i: (0, 0)),
        ],
        out_shape=[
            jax.ShapeDtypeStruct((B, FLAT), jnp.float32),
            jax.ShapeDtypeStruct((B, N), jnp.int32),
            jax.ShapeDtypeStruct((1, 1), jnp.float32),
            jax.ShapeDtypeStruct((1, 1), jnp.float32),
        ],
    )(x2d, We1, row(be1), We2, row(be2), We3, row(be3), codebook, cbT,
      Wd1, row(bd1), Wd2, row(bd2), Wd3, row(bd3))

    cl = vq_sum[0, 0] / (B * N * D)
    vq_loss = cl + CC * cl
    recon_loss = rec_sum[0, 0] / (B * T * A)
    loss = recon_loss + vq_loss
    return (recon2d.reshape(B, T, A), idx, vq_loss, recon_loss, loss)


# speedup vs baseline: 1.2995x; 1.1276x over previous
import functools

import jax
import jax.numpy as jnp
from jax import lax
from jax.experimental import pallas as pl
from jax.experimental.pallas import tpu as pltpu
from jax.experimental.pallas import tpu_sc as plsc

B, T, A = 4096, 50, 3
H, D, N, K = 256, 32, 4, 8192
FLAT = T * A
CC = 0.25
BT = 256


def _silu(x):
    return x * jax.nn.sigmoid(x)


def _mm_bf16(a, b):
    return jnp.dot(a.astype(jnp.bfloat16), b.astype(jnp.bfloat16),
                   preferred_element_type=jnp.float32)


def _mm_f32(a, b):
    return jnp.dot(a, b, preferred_element_type=jnp.float32)


def _enc_kernel(x_ref, We1_ref, be1_ref, We2_ref, be2_ref, We3_ref, be3_ref,
                cbT_ref, enc_ref, idx_ref):
    x = x_ref[...]
    h = _silu(_mm_bf16(x, We1_ref[...]) + be1_ref[...])
    h = _silu(_mm_bf16(h, We2_ref[...]) + be2_ref[...])
    enc = _mm_bf16(h, We3_ref[...]) + be3_ref[...]
    enc_ref[...] = enc

    cbT = cbT_ref[...]
    cn = jnp.sum(cbT * cbT, axis=0, keepdims=True)
    KH = K // 2
    iota_h = jax.lax.broadcasted_iota(jnp.int32, (1, KH), 1)

    idxs = []
    for n in range(N):
        fln = enc[:, n * D:(n + 1) * D]
        ln = jnp.sum(fln * fln, axis=1, keepdims=True)
        m = _mm_f32(fln, cbT)
        dist = (ln + cn) - 2.0 * m
        dA = dist[:, :KH]
        dB = dist[:, KH:]
        mnA = jnp.min(dA, axis=1, keepdims=True)
        amA = jnp.min(jnp.where(dA == mnA, iota_h, K), axis=1, keepdims=True)
        mnB = jnp.min(dB, axis=1, keepdims=True)
        amB = jnp.min(jnp.where(dB == mnB, iota_h, K), axis=1,
                      keepdims=True) + KH
        accA = mnA.astype(jnp.bfloat16).astype(jnp.float32)
        idxs.append(jnp.where(mnB < accA, amB, amA))

    idx_ref[...] = jnp.concatenate(idxs, axis=1)


def _dec_kernel(x_ref, enc_ref, quant_ref, Wd1_ref, bd1_ref, Wd2_ref,
                bd2_ref, Wd3_ref, bd3_ref, recon_ref, vq_ref, rec_ref):
    i = pl.program_id(0)
    quant = quant_ref[...]
    diff = quant - enc_ref[...]
    vq_part = jnp.sum(diff * diff)

    dec = _silu(_mm_bf16(quant, Wd1_ref[...]) + bd1_ref[...])
    dec = _silu(_mm_bf16(dec, Wd2_ref[...]) + bd2_ref[...])
    recon = _mm_bf16(dec, Wd3_ref[...]) + bd3_ref[...]
    recon_ref[...] = recon
    rdiff = recon - x_ref[...]
    rec_part = jnp.sum(rdiff * rdiff)

    @pl.when(i == 0)
    def _init():
        vq_ref[...] = jnp.zeros((1, 1), jnp.float32)
        rec_ref[...] = jnp.zeros((1, 1), jnp.float32)

    vq_ref[...] += vq_part
    rec_ref[...] += rec_part


GW = 128


def _sc_gather(table_padded, idx_flat):
    info = plsc.get_sparse_core_info()
    nw = info.num_cores * info.num_subcores
    rows = B * N
    b_per_w = rows // nw
    mesh = plsc.VectorSubcoreMesh(core_axis_name="c", subcore_axis_name="s")

    @functools.partial(
        pl.kernel, mesh=mesh,
        out_type=jax.ShapeDtypeStruct((rows, GW), jnp.float32),
        scratch_types=[
            pltpu.VMEM((b_per_w,), jnp.int32),
            pltpu.VMEM((b_per_w, GW), jnp.float32),
            pltpu.SemaphoreType.DMA,
        ],
    )
    def k(table_hbm, idx_hbm, out_hbm, idx_v, rows_v, sem):
        wid = lax.axis_index("s") * info.num_cores + lax.axis_index("c")
        base = wid * b_per_w
        pltpu.sync_copy(idx_hbm.at[pl.ds(base, b_per_w)], idx_v)
        pltpu.async_copy(table_hbm.at[idx_v], rows_v, sem).wait()
        pltpu.sync_copy(rows_v, out_hbm.at[pl.ds(base, b_per_w)])

    return k(table_padded, idx_flat)


def kernel(actions, We1, be1, We2, be2, We3, be3, codebook, Wd1, bd1, Wd2,
           bd2, Wd3, bd3):
    x2d = actions.reshape(B, FLAT)
    cbT = codebook.T
    row = lambda v: v.reshape(1, -1)
    full = lambda shape: pl.BlockSpec(shape, lambda i: (0, 0))
    grid = (B // BT,)

    enc, idx = pl.pallas_call(
        _enc_kernel,
        grid=grid,
        in_specs=[
            pl.BlockSpec((BT, FLAT), lambda i: (i, 0)),
            full((FLAT, H)), full((1, H)),
            full((H, H)), full((1, H)),
            full((H, N * D)), full((1, N * D)),
            full((D, K)),
        ],
        out_specs=[
            pl.BlockSpec((BT, N * D), lambda i: (i, 0)),
            pl.BlockSpec((BT, N), lambda i: (i, 0)),
        ],
        out_shape=[
            jax.ShapeDtypeStruct((B, N * D), jnp.float32),
            jax.ShapeDtypeStruct((B, N), jnp.int32),
        ],
    )(x2d, We1, row(be1), We2, row(be2), We3, row(be3), cbT)

    cb_pad = jnp.pad(codebook, ((0, 0), (0, GW - D)))
    gathered = _sc_gather(cb_pad, idx.reshape(B * N))
    quant = gathered[:, :D]

    recon2d, vq_sum, rec_sum = pl.pallas_call(
        _dec_kernel,
        grid=grid,
        in_specs=[
            pl.BlockSpec((BT, FLAT), lambda i: (i, 0)),
            pl.BlockSpec((BT, N * D), lambda i: (i, 0)),
            pl.BlockSpec((BT, N * D), lambda i: (i, 0)),
            full((N * D, H)), full((1, H)),
            full((H, H)), full((1, H)),
            full((H, FLAT)), full((1, FLAT)),
        ],
        out_specs=[
            pl.BlockSpec((BT, FLAT), lambda i: (i, 0)),
            pl.BlockSpec((1, 1), lambda i: (0, 0)),
            pl.BlockSpec((1, 1), lambda i: (0, 0)),
        ],
        out_shape=[
            jax.ShapeDtypeStruct((B, FLAT), jnp.float32),
            jax.ShapeDtypeStruct((1, 1), jnp.float32),
            jax.ShapeDtypeStruct((1, 1), jnp.float32),
        ],
    )(x2d, enc, quant.reshape(B, N * D),
      Wd1, row(bd1), Wd2, row(bd2), Wd3, row(bd3))

    cl = vq_sum[0, 0] / (B * N * D)
    vq_loss = cl + CC * cl
    recon_loss = rec_sum[0, 0] / (B * T * A)
    loss = recon_loss + vq_loss
    return (recon2d.reshape(B, T, A), idx, vq_loss, recon_loss, loss)


# speedup vs baseline: 1.3205x; 1.0162x over previous
import functools

import jax
import jax.numpy as jnp
from jax import lax
from jax.experimental import pallas as pl
from jax.experimental.pallas import tpu as pltpu
from jax.experimental.pallas import tpu_sc as plsc

B, T, A = 4096, 50, 3
H, D, N, K = 256, 32, 4, 8192
FLAT = T * A
CC = 0.25
BT = 256


def _silu(x):
    return x * jax.nn.sigmoid(x)


def _mm_bf16(a, b):
    return jnp.dot(a.astype(jnp.bfloat16), b.astype(jnp.bfloat16),
                   preferred_element_type=jnp.float32)


def _mm_f32(a, b):
    return jnp.dot(a, b, preferred_element_type=jnp.float32)


def _enc_kernel(x_ref, We1_ref, be1_ref, We2_ref, be2_ref, We3_ref, be3_ref,
                cbT_ref, enc_ref, idx_ref):
    x = x_ref[...]
    h = _silu(_mm_bf16(x, We1_ref[...]) + be1_ref[...])
    h = _silu(_mm_bf16(h, We2_ref[...]) + be2_ref[...])
    enc = _mm_bf16(h, We3_ref[...]) + be3_ref[...]
    enc_ref[...] = enc

    cbT = cbT_ref[...]
    cn = jnp.sum(cbT * cbT, axis=0, keepdims=True)
    KH = K // 2
    iota_h = jax.lax.broadcasted_iota(jnp.int32, (1, KH), 1)

    idxs = []
    for n in range(N):
        fln = enc[:, n * D:(n + 1) * D]
        ln = jnp.sum(fln * fln, axis=1, keepdims=True)
        m = _mm_f32(fln, cbT)
        dist = (ln + cn) - 2.0 * m
        dA = dist[:, :KH]
        dB = dist[:, KH:]
        mnA = jnp.min(dA, axis=1, keepdims=True)
        amA = jnp.min(jnp.where(dA == mnA, iota_h, K), axis=1, keepdims=True)
        mnB = jnp.min(dB, axis=1, keepdims=True)
        amB = jnp.min(jnp.where(dB == mnB, iota_h, K), axis=1,
                      keepdims=True) + KH
        accA = mnA.astype(jnp.bfloat16).astype(jnp.float32)
        idxs.append(jnp.where(mnB < accA, amB, amA))

    idx_ref[...] = jnp.concatenate(idxs, axis=1)


def _dec_kernel(x_ref, enc_ref, quant_ref, Wd1_ref, bd1_ref, Wd2_ref,
                bd2_ref, Wd3_ref, bd3_ref, recon_ref, vq_ref, rec_ref):
    quant = quant_ref[...]
    diff = quant - enc_ref[...]
    vq_ref[...] = jnp.broadcast_to(jnp.sum(diff * diff), (1, 1, 128))

    dec = _silu(_mm_bf16(quant, Wd1_ref[...]) + bd1_ref[...])
    dec = _silu(_mm_bf16(dec, Wd2_ref[...]) + bd2_ref[...])
    recon = _mm_bf16(dec, Wd3_ref[...]) + bd3_ref[...]
    recon_ref[...] = recon
    rdiff = recon - x_ref[...]
    rec_ref[...] = jnp.broadcast_to(jnp.sum(rdiff * rdiff), (1, 1, 128))


GW = 128


def _sc_gather(table_padded, idx_flat):
    info = plsc.get_sparse_core_info()
    nw = info.num_cores * info.num_subcores
    rows = B * N
    b_per_w = rows // nw
    mesh = plsc.VectorSubcoreMesh(core_axis_name="c", subcore_axis_name="s")

    @functools.partial(
        pl.kernel, mesh=mesh,
        out_type=jax.ShapeDtypeStruct((rows, GW), jnp.float32),
        scratch_types=[
            pltpu.VMEM((b_per_w,), jnp.int32),
            pltpu.VMEM((b_per_w, GW), jnp.float32),
            pltpu.SemaphoreType.DMA,
        ],
    )
    def k(table_hbm, idx_hbm, out_hbm, idx_v, rows_v, sem):
        wid = lax.axis_index("s") * info.num_cores + lax.axis_index("c")
        base = wid * b_per_w
        pltpu.sync_copy(idx_hbm.at[pl.ds(base, b_per_w)], idx_v)
        pltpu.async_copy(table_hbm.at[idx_v], rows_v, sem).wait()
        pltpu.sync_copy(rows_v, out_hbm.at[pl.ds(base, b_per_w)])

    return k(table_padded, idx_flat)


def kernel(actions, We1, be1, We2, be2, We3, be3, codebook, Wd1, bd1, Wd2,
           bd2, Wd3, bd3):
    x2d = actions.reshape(B, FLAT)
    cbT = codebook.T
    row = lambda v: v.reshape(1, -1)
    full = lambda shape: pl.BlockSpec(shape, lambda i: (0, 0))
    grid = (B // BT,)

    enc, idx = pl.pallas_call(
        _enc_kernel,
        grid=grid,
        in_specs=[
            pl.BlockSpec((BT, FLAT), lambda i: (i, 0)),
            full((FLAT, H)), full((1, H)),
            full((H, H)), full((1, H)),
            full((H, N * D)), full((1, N * D)),
            full((D, K)),
        ],
        out_specs=[
            pl.BlockSpec((BT, N * D), lambda i: (i, 0)),
            pl.BlockSpec((BT, N), lambda i: (i, 0)),
        ],
        out_shape=[
            jax.ShapeDtypeStruct((B, N * D), jnp.float32),
            jax.ShapeDtypeStruct((B, N), jnp.int32),
        ],
        compiler_params=pltpu.CompilerParams(
            dimension_semantics=("parallel",)),
    )(x2d, We1, row(be1), We2, row(be2), We3, row(be3), cbT)

    cb_pad = jnp.pad(codebook, ((0, 0), (0, GW - D)))
    gathered = _sc_gather(cb_pad, idx.reshape(B * N))
    quant = gathered[:, :D]

    recon2d, vq_sum, rec_sum = pl.pallas_call(
        _dec_kernel,
        grid=grid,
        in_specs=[
            pl.BlockSpec((BT, FLAT), lambda i: (i, 0)),
            pl.BlockSpec((BT, N * D), lambda i: (i, 0)),
            pl.BlockSpec((BT, N * D), lambda i: (i, 0)),
            full((N * D, H)), full((1, H)),
            full((H, H)), full((1, H)),
            full((H, FLAT)), full((1, FLAT)),
        ],
        out_specs=[
            pl.BlockSpec((BT, FLAT), lambda i: (i, 0)),
            pl.BlockSpec((1, 1, 128), lambda i: (i, 0, 0)),
            pl.BlockSpec((1, 1, 128), lambda i: (i, 0, 0)),
        ],
        out_shape=[
            jax.ShapeDtypeStruct((B, FLAT), jnp.float32),
            jax.ShapeDtypeStruct((B // BT, 1, 128), jnp.float32),
            jax.ShapeDtypeStruct((B // BT, 1, 128), jnp.float32),
        ],
        compiler_params=pltpu.CompilerParams(
            dimension_semantics=("parallel",)),
    )(x2d, enc, quant.reshape(B, N * D),
      Wd1, row(bd1), Wd2, row(bd2), Wd3, row(bd3))

    cl = jnp.sum(vq_sum[:, 0, 0]) / (B * N * D)
    vq_loss = cl + CC * cl
    recon_loss = jnp.sum(rec_sum[:, 0, 0]) / (B * T * A)
    loss = recon_loss + vq_loss
    return (recon2d.reshape(B, T, A), idx, vq_loss, recon_loss, loss)


# speedup vs baseline: 1.3317x; 1.0085x over previous
import functools

import jax
import jax.numpy as jnp
from jax import lax
from jax.experimental import pallas as pl
from jax.experimental.pallas import tpu as pltpu
from jax.experimental.pallas import tpu_sc as plsc

B, T, A = 4096, 50, 3
H, D, N, K = 256, 32, 4, 8192
FLAT = T * A
CC = 0.25
BT = 256
BTE = 512


def _silu(x):
    return x * jax.nn.sigmoid(x)


def _mm_bf16(a, b):
    return jnp.dot(a.astype(jnp.bfloat16), b.astype(jnp.bfloat16),
                   preferred_element_type=jnp.float32)


def _mm_f32(a, b):
    return jnp.dot(a, b, preferred_element_type=jnp.float32)


def _enc_kernel(x_ref, We1_ref, be1_ref, We2_ref, be2_ref, We3_ref, be3_ref,
                cbT_ref, enc_ref, idx_ref):
    x = x_ref[...]
    h = _silu(_mm_bf16(x, We1_ref[...]) + be1_ref[...])
    h = _silu(_mm_bf16(h, We2_ref[...]) + be2_ref[...])
    enc = _mm_bf16(h, We3_ref[...]) + be3_ref[...]
    enc_ref[...] = enc

    cbT = cbT_ref[...]
    cn = jnp.sum(cbT * cbT, axis=0, keepdims=True)
    KH = K // 2
    iota_h = jax.lax.broadcasted_iota(jnp.int32, (1, KH), 1)

    idxs = []
    for n in range(N):
        fln = enc[:, n * D:(n + 1) * D]
        ln = jnp.sum(fln * fln, axis=1, keepdims=True)
        m = _mm_f32(fln, cbT)
        dist = (ln + cn) - 2.0 * m
        dA = dist[:, :KH]
        dB = dist[:, KH:]
        mnA = jnp.min(dA, axis=1, keepdims=True)
        amA = jnp.min(jnp.where(dA == mnA, iota_h, K), axis=1, keepdims=True)
        mnB = jnp.min(dB, axis=1, keepdims=True)
        amB = jnp.min(jnp.where(dB == mnB, iota_h, K), axis=1,
                      keepdims=True) + KH
        accA = mnA.astype(jnp.bfloat16).astype(jnp.float32)
        idxs.append(jnp.where(mnB < accA, amB, amA))

    idx_ref[...] = jnp.concatenate(idxs, axis=1)


def _dec_kernel(x_ref, enc_ref, quant_ref, Wd1_ref, bd1_ref, Wd2_ref,
                bd2_ref, Wd3_ref, bd3_ref, recon_ref, vq_ref, rec_ref):
    quant = quant_ref[...]
    diff = quant - enc_ref[...]
    vq_ref[...] = jnp.broadcast_to(jnp.sum(diff * diff), (1, 1, 128))

    dec = _silu(_mm_bf16(quant, Wd1_ref[...]) + bd1_ref[...])
    dec = _silu(_mm_bf16(dec, Wd2_ref[...]) + bd2_ref[...])
    recon = _mm_bf16(dec, Wd3_ref[...]) + bd3_ref[...]
    recon_ref[...] = recon
    rdiff = recon - x_ref[...]
    rec_ref[...] = jnp.broadcast_to(jnp.sum(rdiff * rdiff), (1, 1, 128))


GW = 128


def _sc_gather(table_padded, idx_flat):
    info = plsc.get_sparse_core_info()
    nw = info.num_cores * info.num_subcores
    rows = B * N
    b_per_w = rows // nw
    mesh = plsc.VectorSubcoreMesh(core_axis_name="c", subcore_axis_name="s")

    @functools.partial(
        pl.kernel, mesh=mesh,
        out_type=jax.ShapeDtypeStruct((rows, GW), jnp.float32),
        scratch_types=[
            pltpu.VMEM((b_per_w,), jnp.int32),
            pltpu.VMEM((b_per_w, GW), jnp.float32),
            pltpu.SemaphoreType.DMA,
        ],
    )
    def k(table_hbm, idx_hbm, out_hbm, idx_v, rows_v, sem):
        wid = lax.axis_index("s") * info.num_cores + lax.axis_index("c")
        base = wid * b_per_w
        pltpu.sync_copy(idx_hbm.at[pl.ds(base, b_per_w)], idx_v)
        pltpu.async_copy(table_hbm.at[idx_v], rows_v, sem).wait()
        pltpu.sync_copy(rows_v, out_hbm.at[pl.ds(base, b_per_w)])

    return k(table_padded, idx_flat)


def kernel(actions, We1, be1, We2, be2, We3, be3, codebook, Wd1, bd1, Wd2,
           bd2, Wd3, bd3):
    x2d = actions.reshape(B, FLAT)
    cbT = codebook.T
    row = lambda v: v.reshape(1, -1)
    full = lambda shape: pl.BlockSpec(shape, lambda i: (0, 0))
    grid = (B // BT,)

    enc, idx = pl.pallas_call(
        _enc_kernel,
        grid=(B // BTE,),
        in_specs=[
            pl.BlockSpec((BTE, FLAT), lambda i: (i, 0)),
            full((FLAT, H)), full((1, H)),
            full((H, H)), full((1, H)),
            full((H, N * D)), full((1, N * D)),
            full((D, K)),
        ],
        out_specs=[
            pl.BlockSpec((BTE, N * D), lambda i: (i, 0)),
            pl.BlockSpec((BTE, N), lambda i: (i, 0)),
        ],
        out_shape=[
            jax.ShapeDtypeStruct((B, N * D), jnp.float32),
            jax.ShapeDtypeStruct((B, N), jnp.int32),
        ],
        compiler_params=pltpu.CompilerParams(
            dimension_semantics=("parallel",)),
    )(x2d, We1, row(be1), We2, row(be2), We3, row(be3), cbT)

    cb_pad = jnp.pad(codebook, ((0, 0), (0, GW - D)))
    gathered = _sc_gather(cb_pad, idx.reshape(B * N))
    quant = gathered[:, :D]

    recon2d, vq_sum, rec_sum = pl.pallas_call(
        _dec_kernel,
        grid=grid,
        in_specs=[
            pl.BlockSpec((BT, FLAT), lambda i: (i, 0)),
            pl.BlockSpec((BT, N * D), lambda i: (i, 0)),
            pl.BlockSpec((BT, N * D), lambda i: (i, 0)),
            full((N * D, H)), full((1, H)),
            full((H, H)), full((1, H)),
            full((H, FLAT)), full((1, FLAT)),
        ],
        out_specs=[
            pl.BlockSpec((BT, FLAT), lambda i: (i, 0)),
            pl.BlockSpec((1, 1, 128), lambda i: (i, 0, 0)),
            pl.BlockSpec((1, 1, 128), lambda i: (i, 0, 0)),
        ],
        out_shape=[
            jax.ShapeDtypeStruct((B, FLAT), jnp.float32),
            jax.ShapeDtypeStruct((B // BT, 1, 128), jnp.float32),
            jax.ShapeDtypeStruct((B // BT, 1, 128), jnp.float32),
        ],
        compiler_params=pltpu.CompilerParams(
            dimension_semantics=("parallel",)),
    )(x2d, enc, quant.reshape(B, N * D),
      Wd1, row(bd1), Wd2, row(bd2), Wd3, row(bd3))

    cl = jnp.sum(vq_sum[:, 0, 0]) / (B * N * D)
    vq_loss = cl + CC * cl
    recon_loss = jnp.sum(rec_sum[:, 0, 0]) / (B * T * A)
    loss = recon_loss + vq_loss
    return (recon2d.reshape(B, T, A), idx, vq_loss, recon_loss, loss)
